# baseline (speedup 1.0000x reference)
import functools

import jax
import jax.numpy as jnp
from jax import lax
from jax.experimental import pallas as pl
from jax.experimental.pallas import tpu as pltpu
from jax.experimental.pallas import tpu_sc as plsc

BATCH = 16384
EMBED = 50
LANES = 16


def _body(uid_hbm, aid_hbm, utab_hbm, atab_hbm, w_hbm, b_hbm, out_hbm,
          uidx_v, aidx_v, urows_v, arows_v, w_v, b_v, out_v, usem, asem,
          *, b_per_w, num_cores):
  wid = lax.axis_index("s") * num_cores + lax.axis_index("c")
  base = wid * b_per_w

  pltpu.sync_copy(uid_hbm.at[pl.ds(base, b_per_w)], uidx_v)
  pltpu.sync_copy(aid_hbm.at[pl.ds(base, b_per_w)], aidx_v)
  pltpu.sync_copy(w_hbm, w_v)
  pltpu.sync_copy(b_hbm, b_v)

  cu = pltpu.async_copy(utab_hbm.at[uidx_v], urows_v, usem)
  ca = pltpu.async_copy(atab_hbm.at[aidx_v], arows_v, asem)
  cu.wait()
  ca.wait()

  w = w_v[...]
  b = b_v[...]
  lane = lax.iota(jnp.int32, LANES)

  def group(g, carry):
    rows = g * LANES + lane
    acc = jnp.zeros((LANES,), jnp.float32)
    for d in range(EMBED):
      dv = jnp.full((LANES,), d, jnp.int32)
      uv = plsc.load_gather(urows_v, [rows, dv])
      av = plsc.load_gather(arows_v, [rows, dv])
      acc = acc + uv * av
    z = acc * w + b
    res = 1.0 / (1.0 + jnp.exp(-z))
    out_v[pl.ds(g * LANES, LANES)] = res
    return carry

  lax.fori_loop(0, b_per_w // LANES, group, 0)
  pltpu.sync_copy(out_v, out_hbm.at[pl.ds(base, b_per_w)])


def kernel(user_id, ad_id, user_table, ad_table, fc_w, fc_b):
  info = plsc.get_sparse_core_info()
  nc, ns = info.num_cores, info.num_subcores
  nw = nc * ns
  b_per_w = BATCH // nw

  scale = jnp.full((LANES,), fc_w[0, 0], jnp.float32)
  bias = jnp.full((LANES,), fc_b[0], jnp.float32)

  mesh = plsc.VectorSubcoreMesh(core_axis_name="c", subcore_axis_name="s")
  k = pl.kernel(
      functools.partial(_body, b_per_w=b_per_w, num_cores=nc),
      out_type=jax.ShapeDtypeStruct((BATCH,), jnp.float32),
      mesh=mesh,
      compiler_params=pltpu.CompilerParams(
          needs_layout_passes=False, use_tc_tiling_on_sc=False),
      scratch_types=[
          pltpu.VMEM((b_per_w,), jnp.int32),
          pltpu.VMEM((b_per_w,), jnp.int32),
          pltpu.VMEM((b_per_w, EMBED), jnp.float32),
          pltpu.VMEM((b_per_w, EMBED), jnp.float32),
          pltpu.VMEM((LANES,), jnp.float32),
          pltpu.VMEM((LANES,), jnp.float32),
          pltpu.VMEM((b_per_w,), jnp.float32),
          pltpu.SemaphoreType.DMA,
          pltpu.SemaphoreType.DMA,
      ],
      name="ad_user_embedding_sc",
  )
  out = k(user_id.astype(jnp.int32), ad_id.astype(jnp.int32),
          user_table, ad_table, scale, bias)
  return out.reshape(BATCH, 1)
